# baseline (speedup 1.0000x reference)
import functools

import jax
import jax.numpy as jnp
import numpy as np
from jax import lax
from jax.experimental import pallas as pl
from jax.experimental.pallas import tpu as pltpu
from jax.experimental.pallas import tpu_sc as plsc

N_DIM = 512
N_KEYS = 512
HEADS = 4
KNN = 32
HALF = N_DIM // 2
N_TOK = 8192
TB = 128
NCAND = 128

_PAIRS = [(a, b) for a in range(KNN) for b in range(KNN) if (a + 1) * (b + 1) <= KNN]
_PAIRS = _PAIRS + [(KNN - 1, KNN - 1)] * (NCAND - len(_PAIRS))
_SELA = np.zeros((NCAND, KNN), np.float32)
_SELB = np.zeros((NCAND, KNN), np.float32)
for _c, (_a, _b) in enumerate(_PAIRS):
    _SELA[_c, _a] = 1.0
    _SELB[_c, _b] = 1.0


def _topk_T(s, k):
    n = s.shape[0]
    iota = lax.broadcasted_iota(jnp.int32, s.shape, 0)
    vals, idxs = [], []
    for _ in range(k):
        m = jnp.max(s, axis=0)
        sel = s == m[None, :]
        idx = jnp.min(jnp.where(sel, iota, n), axis=0)
        s = jnp.where(sel, -jnp.inf, s)
        vals.append(m)
        idxs.append(idx)
    return jnp.stack(vals), jnp.stack(idxs)


def _select_body(wq_ref, keys_ref, x_ref, selA_ref, selB_ref, idx_ref, w_ref):
    selA = selA_ref[...]
    selB = selB_ref[...]
    xb = x_ref[...].astype(jnp.bfloat16)
    hi = lax.Precision.HIGHEST
    qT = lax.dot_general(wq_ref[...], xb, (((1,), (1,)), ((), ())),
                         preferred_element_type=jnp.float32)
    qT16 = qT.astype(jnp.bfloat16)
    for u in range(HEADS):
        q1 = qT16[u * N_DIM: u * N_DIM + HALF, :]
        q2 = qT16[u * N_DIM + HALF: (u + 1) * N_DIM, :]
        sT1 = lax.dot_general(keys_ref[u], q1, (((1,), (0,)), ((), ())),
                              preferred_element_type=jnp.float32)
        sT2 = lax.dot_general(keys_ref[HEADS + u], q2, (((1,), (0,)), ((), ())),
                              preferred_element_type=jnp.float32)
        sc1, i1 = _topk_T(sT1, KNN)
        sc2, i2 = _topk_T(sT2, KNN)
        cand = (jnp.dot(selA, sc1, precision=hi)
                + jnp.dot(selB, sc2, precision=hi))
        ci1 = jnp.dot(selA, i1.astype(jnp.float32), precision=hi)
        ci2 = jnp.dot(selB, i2.astype(jnp.float32), precision=hi)
        cidx = ci1 * float(2 * N_KEYS) + ci2 * 2.0
        for k in range(KNN):
            m = jnp.max(cand, axis=0)
            sel = cand == m[None, :]
            iv = jnp.max(jnp.where(sel, cidx, -1.0), axis=0)
            cand = jnp.where(sel, -jnp.inf, cand)
            idx_ref[u * KNN + k, :] = iv
            w_ref[u * KNN + k, :] = jnp.maximum(m, 0.0)


_NC = 2
_NS = 16
_NW = _NC * _NS
_NTW = _NW // 2
_TPW = N_TOK // _NTW
_GRP = 16
_NGRP = _TPW // _GRP
_CHUNK = 64
_CPT = (HEADS * KNN) // _CHUNK
_NCH = _GRP * _CPT
_CH = N_DIM // 2
_CSL = _CH // 16


def _full16(v):
    return jnp.full((16,), v, jnp.int32)


def _emb_bag_body(values2, idx2, w, out, idx_v, w_v, rows_v, out_v, gsem, osem):
    wid = lax.axis_index("s") * _NC + lax.axis_index("c")
    gw = wid // 2
    chalf = wid % 2
    tok0_w = gw * _TPW
    chunk_row0 = gw * (_TPW * _CPT)

    def start(j):
        pltpu.async_copy(values2.at[idx_v.at[j]], rows_v.at[j % 2],
                         gsem.at[j % 2])

    def wait_g(j):
        pltpu.make_async_copy(values2.at[idx_v.at[j]], rows_v.at[j % 2],
                              gsem.at[j % 2]).wait()

    def group_body(g, _):
        pltpu.sync_copy(
            idx2.at[pl.ds(chunk_row0 + g * _NCH, _NCH)], idx_v)

        def fix_idx(r, carry):
            for q4 in range(_CHUNK // 16):
                sl = pl.ds(q4 * 16, 16)
                idx_v[r, sl] = idx_v[r, sl] + chalf
            return carry

        lax.fori_loop(0, _NCH, fix_idx, 0)
        pltpu.sync_copy(
            w.at[pl.ds((tok0_w + g * _GRP) * (HEADS * KNN),
                       _GRP * HEADS * KNN)], w_v)
        obuf = g % 2

        @pl.when(g >= 2)
        def _():
            pltpu.make_async_copy(
                out_v.at[obuf],
                out.at[pl.ds(tok0_w + (g - 2) * _GRP, _GRP),
                       pl.ds(chalf * _CH, _CH)],
                osem.at[obuf]).wait()

        start(0)
        start(1)

        def tok_body(t, _):
            dn = lax.GatherDimensionNumbers(
                offset_dims=(), collapsed_slice_dims=(0,),
                start_index_map=(0,))
            zero = jnp.zeros((16,), jnp.float32)
            for c in range(_CSL):
                out_v[obuf, t, pl.ds(c * 16, 16)] = zero
            for half in range(_CPT):
                j = _CPT * t + half
                wait_g(j)
                b = j % 2

                def blk_body(k16, carry, half=half, b=b):
                    wv16 = w_v[pl.ds(
                        t * (HEADS * KNN) + half * _CHUNK + k16 * 16, 16)]
                    acc = [zero] * _CSL
                    for jj in range(16):
                        wk = lax.gather(
                            wv16, _full16(jj)[:, None], dn, (1,),
                            mode=lax.GatherScatterMode.PROMISE_IN_BOUNDS)
                        k = k16 * 16 + jj
                        for c in range(_CSL):
                            acc[c] = acc[c] + wk * rows_v[b, k,
                                                          pl.ds(c * 16, 16)]
                    for c in range(_CSL):
                        sl = pl.ds(c * 16, 16)
                        out_v[obuf, t, sl] = out_v[obuf, t, sl] + acc[c]
                    return carry

                lax.fori_loop(0, _CHUNK // 16, blk_body, 0)

                @pl.when(j + 2 < _NCH)
                def _():
                    start(j + 2)
            return 0

        lax.fori_loop(0, _GRP, tok_body, 0)
        pltpu.async_copy(
            out_v.at[obuf],
            out.at[pl.ds(tok0_w + g * _GRP, _GRP), pl.ds(chalf * _CH, _CH)],
            osem.at[obuf])
        return 0

    lax.fori_loop(0, _NGRP, group_body, 0)
    for g in (_NGRP - 2, _NGRP - 1):
        pltpu.make_async_copy(
            out_v.at[g % 2],
            out.at[pl.ds(tok0_w + g * _GRP, _GRP), pl.ds(chalf * _CH, _CH)],
            osem.at[g % 2]).wait()


def _emb_bag(values, idx2, w):
    values2 = values.reshape(2 * values.shape[0], _CH)
    mesh = plsc.VectorSubcoreMesh(core_axis_name="c", subcore_axis_name="s")
    kern = functools.partial(
        pl.kernel, mesh=mesh,
        out_type=jax.ShapeDtypeStruct((N_TOK, N_DIM), jnp.float32),
        scratch_types=[
            pltpu.VMEM((_NCH, _CHUNK), jnp.int32),
            pltpu.VMEM((_GRP * HEADS * KNN,), jnp.float32),
            pltpu.VMEM((2, _CHUNK, _CH), jnp.float32),
            pltpu.VMEM((2, _GRP, _CH), jnp.float32),
            pltpu.SemaphoreType.DMA((2,)),
            pltpu.SemaphoreType.DMA((2,)),
        ],
    )(_emb_bag_body)
    return kern(values2, idx2, w)


def kernel(x, Wq, keys1, keys2, values):
    keysC = jnp.concatenate([keys1, keys2], axis=0).astype(jnp.bfloat16)
    idxT, wT = pl.pallas_call(
        _select_body,
        grid=(N_TOK // TB,),
        in_specs=[pl.BlockSpec((HEADS * N_DIM, N_DIM), lambda i: (0, 0)),
                  pl.BlockSpec((2 * HEADS, N_KEYS, HALF), lambda i: (0, 0, 0)),
                  pl.BlockSpec((TB, N_DIM), lambda i: (i, 0)),
                  pl.BlockSpec((NCAND, KNN), lambda i: (0, 0)),
                  pl.BlockSpec((NCAND, KNN), lambda i: (0, 0))],
        out_specs=[pl.BlockSpec((HEADS * KNN, TB), lambda i: (0, i)),
                   pl.BlockSpec((HEADS * KNN, TB), lambda i: (0, i))],
        out_shape=[jax.ShapeDtypeStruct((HEADS * KNN, N_TOK), jnp.float32),
                   jax.ShapeDtypeStruct((HEADS * KNN, N_TOK), jnp.float32)],
    )(Wq.astype(jnp.bfloat16), keysC, x,
      jnp.asarray(_SELA), jnp.asarray(_SELB))

    flat_idx = idxT.T.astype(jnp.int32).reshape(N_TOK * _CPT, _CHUNK)
    flat_w = wT.T.reshape(N_TOK * HEADS * KNN)
    return _emb_bag(values, flat_idx, flat_w)


# speedup vs baseline: 1.2367x; 1.2367x over previous
import functools

import jax
import jax.numpy as jnp
import numpy as np
from jax import lax
from jax.experimental import pallas as pl
from jax.experimental.pallas import tpu as pltpu
from jax.experimental.pallas import tpu_sc as plsc

N_DIM = 512
N_KEYS = 512
HEADS = 4
KNN = 32
HALF = N_DIM // 2
N_TOK = 8192
TB = 128
NCAND = 128

_PAIRS = [(a, b) for a in range(KNN) for b in range(KNN) if (a + 1) * (b + 1) <= KNN]
_PAIRS = _PAIRS + [(KNN - 1, KNN - 1)] * (NCAND - len(_PAIRS))
_SELA = np.zeros((NCAND, KNN), np.float32)
_SELB = np.zeros((NCAND, KNN), np.float32)
for _c, (_a, _b) in enumerate(_PAIRS):
    _SELA[_c, _a] = 1.0
    _SELB[_c, _b] = 1.0


def _topk_T(s, k):
    n = s.shape[0]
    iota = lax.broadcasted_iota(jnp.int32, s.shape, 0)
    vals, idxs = [], []
    for _ in range(k):
        m = jnp.max(s, axis=0)
        sel = s == m[None, :]
        idx = jnp.min(jnp.where(sel, iota, n), axis=0)
        s = jnp.where(sel, -jnp.inf, s)
        vals.append(m)
        idxs.append(idx)
    return jnp.stack(vals), jnp.stack(idxs)


def _select_body(wq_ref, keys_ref, x_ref, selA_ref, selB_ref, idx_ref, w_ref):
    selA = selA_ref[...]
    selB = selB_ref[...]
    xb = x_ref[...].astype(jnp.bfloat16)
    hi = lax.Precision.HIGHEST
    qT = lax.dot_general(wq_ref[...], xb, (((1,), (1,)), ((), ())),
                         preferred_element_type=jnp.float32)
    qT16 = qT.astype(jnp.bfloat16)
    for u in range(HEADS):
        q1 = qT16[u * N_DIM: u * N_DIM + HALF, :]
        q2 = qT16[u * N_DIM + HALF: (u + 1) * N_DIM, :]
        sT1 = lax.dot_general(keys_ref[u], q1, (((1,), (0,)), ((), ())),
                              preferred_element_type=jnp.float32)
        sT2 = lax.dot_general(keys_ref[HEADS + u], q2, (((1,), (0,)), ((), ())),
                              preferred_element_type=jnp.float32)
        sc1, i1 = _topk_T(sT1, KNN)
        sc2, i2 = _topk_T(sT2, KNN)
        cand = (jnp.dot(selA, sc1, precision=hi)
                + jnp.dot(selB, sc2, precision=hi))
        ci1 = jnp.dot(selA, i1.astype(jnp.float32), precision=hi)
        ci2 = jnp.dot(selB, i2.astype(jnp.float32), precision=hi)
        cidx = ci1 * float(2 * N_KEYS) + ci2 * 2.0
        for k in range(KNN):
            m = jnp.max(cand, axis=0)
            sel = cand == m[None, :]
            iv = jnp.max(jnp.where(sel, cidx, -1.0), axis=0)
            cand = jnp.where(sel, -jnp.inf, cand)
            idx_ref[u * KNN + k, :] = iv
            w_ref[u * KNN + k, :] = jnp.maximum(m, 0.0)


_NC = 2
_NS = 16
_NW = _NC * _NS
_NTW = _NW // 2
_TPW = N_TOK // _NTW
_GRP = 16
_NGRP = _TPW // _GRP
_CHUNK = 64
_CPT = (HEADS * KNN) // _CHUNK
_NCH = _GRP * _CPT
_CH = N_DIM // 2
_CSL = _CH // 16


def _full16(v):
    return jnp.full((16,), v, jnp.int32)


def _make_emb_bag_body(ntok):
  _TPW = ntok // _NTW
  _NGRP = _TPW // _GRP

  def _emb_bag_body(values2, idx2, w, out, idx_v, w_v, rows_v, out_v, gsem, osem):
    wid = lax.axis_index("s") * _NC + lax.axis_index("c")
    gw = wid // 2
    chalf = wid % 2
    tok0_w = gw * _TPW
    chunk_row0 = gw * (_TPW * _CPT)

    def start(j):
        pltpu.async_copy(values2.at[idx_v.at[j]], rows_v.at[j % 2],
                         gsem.at[j % 2])

    def wait_g(j):
        pltpu.make_async_copy(values2.at[idx_v.at[j]], rows_v.at[j % 2],
                              gsem.at[j % 2]).wait()

    def group_body(g, _):
        pltpu.sync_copy(
            idx2.at[pl.ds(chunk_row0 + g * _NCH, _NCH)], idx_v)

        def fix_idx(r, carry):
            for q4 in range(_CHUNK // 16):
                sl = pl.ds(q4 * 16, 16)
                idx_v[r, sl] = idx_v[r, sl] + chalf
            return carry

        lax.fori_loop(0, _NCH, fix_idx, 0)
        pltpu.sync_copy(
            w.at[pl.ds((tok0_w + g * _GRP) * (HEADS * KNN),
                       _GRP * HEADS * KNN)], w_v)
        obuf = g % 2

        @pl.when(g >= 2)
        def _():
            pltpu.make_async_copy(
                out_v.at[obuf],
                out.at[pl.ds(tok0_w + (g - 2) * _GRP, _GRP),
                       pl.ds(chalf * _CH, _CH)],
                osem.at[obuf]).wait()

        start(0)
        start(1)

        def tok_body(t, _):
            dn = lax.GatherDimensionNumbers(
                offset_dims=(), collapsed_slice_dims=(0,),
                start_index_map=(0,))
            zero = jnp.zeros((16,), jnp.float32)
            for c in range(_CSL):
                out_v[obuf, t, pl.ds(c * 16, 16)] = zero
            for half in range(_CPT):
                j = _CPT * t + half
                wait_g(j)
                b = j % 2

                def blk_body(k16, carry, half=half, b=b):
                    wv16 = w_v[pl.ds(
                        t * (HEADS * KNN) + half * _CHUNK + k16 * 16, 16)]
                    acc = [zero] * _CSL
                    for jj in range(16):
                        wk = lax.gather(
                            wv16, _full16(jj)[:, None], dn, (1,),
                            mode=lax.GatherScatterMode.PROMISE_IN_BOUNDS)
                        k = k16 * 16 + jj
                        for c in range(_CSL):
                            acc[c] = acc[c] + wk * rows_v[b, k,
                                                          pl.ds(c * 16, 16)]
                    for c in range(_CSL):
                        sl = pl.ds(c * 16, 16)
                        out_v[obuf, t, sl] = out_v[obuf, t, sl] + acc[c]
                    return carry

                lax.fori_loop(0, _CHUNK // 16, blk_body, 0)

                @pl.when(j + 2 < _NCH)
                def _():
                    start(j + 2)
            return 0

        lax.fori_loop(0, _GRP, tok_body, 0)
        pltpu.async_copy(
            out_v.at[obuf],
            out.at[pl.ds(tok0_w + g * _GRP, _GRP), pl.ds(chalf * _CH, _CH)],
            osem.at[obuf])
        return 0

    lax.fori_loop(0, _NGRP, group_body, 0)
    for g in (_NGRP - 2, _NGRP - 1):
        pltpu.make_async_copy(
            out_v.at[g % 2],
            out.at[pl.ds(tok0_w + g * _GRP, _GRP), pl.ds(chalf * _CH, _CH)],
            osem.at[g % 2]).wait()

  return _emb_bag_body


def _emb_bag(values2, idx2, w, ntok):
    mesh = plsc.VectorSubcoreMesh(core_axis_name="c", subcore_axis_name="s")
    kern = functools.partial(
        pl.kernel, mesh=mesh,
        out_type=jax.ShapeDtypeStruct((ntok, N_DIM), jnp.float32),
        scratch_types=[
            pltpu.VMEM((_NCH, _CHUNK), jnp.int32),
            pltpu.VMEM((_GRP * HEADS * KNN,), jnp.float32),
            pltpu.VMEM((2, _CHUNK, _CH), jnp.float32),
            pltpu.VMEM((2, _GRP, _CH), jnp.float32),
            pltpu.SemaphoreType.DMA((2,)),
            pltpu.SemaphoreType.DMA((2,)),
        ],
    )(_make_emb_bag_body(ntok))
    return kern(values2, idx2, w)


_NSEG = 2


def kernel(x, Wq, keys1, keys2, values):
    keysC = jnp.concatenate([keys1, keys2], axis=0).astype(jnp.bfloat16)
    wq16 = Wq.astype(jnp.bfloat16)
    selA = jnp.asarray(_SELA)
    selB = jnp.asarray(_SELB)
    values2 = values.reshape(2 * values.shape[0], _CH)
    nseg_tok = N_TOK // _NSEG
    outs = []
    for sgi in range(_NSEG):
        xs = lax.slice_in_dim(x, sgi * nseg_tok, (sgi + 1) * nseg_tok, axis=0)
        idxT, wT = pl.pallas_call(
            _select_body,
            grid=(nseg_tok // TB,),
            in_specs=[pl.BlockSpec((HEADS * N_DIM, N_DIM), lambda i: (0, 0)),
                      pl.BlockSpec((2 * HEADS, N_KEYS, HALF),
                                   lambda i: (0, 0, 0)),
                      pl.BlockSpec((TB, N_DIM), lambda i: (i, 0)),
                      pl.BlockSpec((NCAND, KNN), lambda i: (0, 0)),
                      pl.BlockSpec((NCAND, KNN), lambda i: (0, 0))],
            out_specs=[pl.BlockSpec((HEADS * KNN, TB), lambda i: (0, i)),
                       pl.BlockSpec((HEADS * KNN, TB), lambda i: (0, i))],
            out_shape=[jax.ShapeDtypeStruct((HEADS * KNN, nseg_tok),
                                            jnp.float32),
                       jax.ShapeDtypeStruct((HEADS * KNN, nseg_tok),
                                            jnp.float32)],
        )(wq16, keysC, xs, selA, selB)
        flat_idx = idxT.T.astype(jnp.int32).reshape(nseg_tok * _CPT, _CHUNK)
        flat_w = wT.T.reshape(nseg_tok * HEADS * KNN)
        outs.append(_emb_bag(values2, flat_idx, flat_w, nseg_tok))
    return jnp.concatenate(outs, axis=0)


# speedup vs baseline: 1.3884x; 1.1226x over previous
import functools

import jax
import jax.numpy as jnp
import numpy as np
from jax import lax
from jax.experimental import pallas as pl
from jax.experimental.pallas import tpu as pltpu
from jax.experimental.pallas import tpu_sc as plsc

N_DIM = 512
N_KEYS = 512
HEADS = 4
KNN = 32
HALF = N_DIM // 2
N_TOK = 8192
TB = 128
NCAND = 128

_PAIRS = [(a, b) for a in range(KNN) for b in range(KNN) if (a + 1) * (b + 1) <= KNN]
_PAIRS = _PAIRS + [(KNN - 1, KNN - 1)] * (NCAND - len(_PAIRS))
_SELA = np.zeros((NCAND, KNN), np.float32)
_SELB = np.zeros((NCAND, KNN), np.float32)
for _c, (_a, _b) in enumerate(_PAIRS):
    _SELA[_c, _a] = 1.0
    _SELB[_c, _b] = 1.0


def _topk_T(s, k):
    n = s.shape[0]
    iota = lax.broadcasted_iota(jnp.int32, s.shape, 0)
    vals, idxs = [], []
    for _ in range(k):
        m = jnp.max(s, axis=0)
        sel = s == m[None, :]
        idx = jnp.min(jnp.where(sel, iota, n), axis=0)
        s = jnp.where(sel, -jnp.inf, s)
        vals.append(m)
        idxs.append(idx)
    return jnp.stack(vals), jnp.stack(idxs)


def _select_body(wq_ref, keys_ref, x_ref, selA_ref, selB_ref, idx_ref, w_ref):
    selA = selA_ref[...]
    selB = selB_ref[...]
    xb = x_ref[...].astype(jnp.bfloat16)
    hi = lax.Precision.HIGHEST
    qT = lax.dot_general(wq_ref[...], xb, (((1,), (1,)), ((), ())),
                         preferred_element_type=jnp.float32)
    qT16 = qT.astype(jnp.bfloat16)
    for u in range(HEADS):
        q1 = qT16[u * N_DIM: u * N_DIM + HALF, :]
        q2 = qT16[u * N_DIM + HALF: (u + 1) * N_DIM, :]
        sT1 = lax.dot_general(keys_ref[u], q1, (((1,), (0,)), ((), ())),
                              preferred_element_type=jnp.float32)
        sT2 = lax.dot_general(keys_ref[HEADS + u], q2, (((1,), (0,)), ((), ())),
                              preferred_element_type=jnp.float32)
        sc1, i1 = _topk_T(sT1, KNN)
        sc2, i2 = _topk_T(sT2, KNN)
        cand = (jnp.dot(selA, sc1, precision=hi)
                + jnp.dot(selB, sc2, precision=hi))
        ci1 = jnp.dot(selA, i1.astype(jnp.float32), precision=hi)
        ci2 = jnp.dot(selB, i2.astype(jnp.float32), precision=hi)
        cidx = ci1 * float(2 * N_KEYS) + ci2 * 2.0
        for k in range(KNN):
            m = jnp.max(cand, axis=0)
            sel = cand == m[None, :]
            iv = jnp.max(jnp.where(sel, cidx, -1.0), axis=0)
            cand = jnp.where(sel, -jnp.inf, cand)
            idx_ref[u * KNN + k, :] = iv
            w_ref[u * KNN + k, :] = jnp.maximum(m, 0.0)


_NC = 2
_NS = 16
_NW = _NC * _NS
_NTW = _NW // 2
_TPW = N_TOK // _NTW
_GRP = 16
_NGRP = _TPW // _GRP
_CHUNK = 64
_CPT = (HEADS * KNN) // _CHUNK
_NCH = _GRP * _CPT
_CH = N_DIM // 2
_CSL = _CH // 16


def _full16(v):
    return jnp.full((16,), v, jnp.int32)


def _make_emb_bag_body(ntok):
  _TPW = ntok // _NTW
  _NGRP = _TPW // _GRP

  def _emb_bag_body(values2, idx2, w, out, idx_v, w_v, rows_v, out_v, gsem, osem):
    wid = lax.axis_index("s") * _NC + lax.axis_index("c")
    gw = wid // 2
    chalf = wid % 2
    tok0_w = gw * _TPW
    chunk_row0 = gw * (_TPW * _CPT)

    def start(j):
        pltpu.async_copy(values2.at[idx_v.at[j]], rows_v.at[j % 2],
                         gsem.at[j % 2])

    def wait_g(j):
        pltpu.make_async_copy(values2.at[idx_v.at[j]], rows_v.at[j % 2],
                              gsem.at[j % 2]).wait()

    def group_body(g, _):
        pltpu.sync_copy(
            idx2.at[pl.ds(chunk_row0 + g * _NCH, _NCH)], idx_v)

        def fix_idx(r, carry):
            for q4 in range(_CHUNK // 16):
                sl = pl.ds(q4 * 16, 16)
                idx_v[r, sl] = idx_v[r, sl] + chalf
            return carry

        lax.fori_loop(0, _NCH, fix_idx, 0)
        pltpu.sync_copy(
            w.at[pl.ds((tok0_w + g * _GRP) * (HEADS * KNN),
                       _GRP * HEADS * KNN)], w_v)
        obuf = g % 2

        @pl.when(g >= 2)
        def _():
            pltpu.make_async_copy(
                out_v.at[obuf],
                out.at[pl.ds(tok0_w + (g - 2) * _GRP, _GRP),
                       pl.ds(chalf * _CH, _CH)],
                osem.at[obuf]).wait()

        start(0)
        start(1)

        def tok_body(t, _):
            dn = lax.GatherDimensionNumbers(
                offset_dims=(), collapsed_slice_dims=(0,),
                start_index_map=(0,))
            zero = jnp.zeros((16,), jnp.float32)
            for c in range(_CSL):
                out_v[obuf, t, pl.ds(c * 16, 16)] = zero
            for half in range(_CPT):
                j = _CPT * t + half
                wait_g(j)
                b = j % 2

                def blk_body(k16, carry, half=half, b=b):
                    wv16 = w_v[pl.ds(
                        t * (HEADS * KNN) + half * _CHUNK + k16 * 16, 16)]
                    acc = [zero] * _CSL
                    for jj in range(16):
                        wk = lax.gather(
                            wv16, _full16(jj)[:, None], dn, (1,),
                            mode=lax.GatherScatterMode.PROMISE_IN_BOUNDS)
                        k = k16 * 16 + jj
                        for c in range(_CSL):
                            acc[c] = acc[c] + wk * rows_v[b, k,
                                                          pl.ds(c * 16, 16)]
                    for c in range(_CSL):
                        sl = pl.ds(c * 16, 16)
                        out_v[obuf, t, sl] = out_v[obuf, t, sl] + acc[c]
                    return carry

                lax.fori_loop(0, _CHUNK // 16, blk_body, 0)

                @pl.when(j + 2 < _NCH)
                def _():
                    start(j + 2)
            return 0

        lax.fori_loop(0, _GRP, tok_body, 0)
        pltpu.async_copy(
            out_v.at[obuf],
            out.at[pl.ds(tok0_w + g * _GRP, _GRP), pl.ds(chalf * _CH, _CH)],
            osem.at[obuf])
        return 0

    lax.fori_loop(0, _NGRP, group_body, 0)
    for g in (_NGRP - 2, _NGRP - 1):
        pltpu.make_async_copy(
            out_v.at[g % 2],
            out.at[pl.ds(tok0_w + g * _GRP, _GRP), pl.ds(chalf * _CH, _CH)],
            osem.at[g % 2]).wait()

  return _emb_bag_body


def _emb_bag(values2, idx2, w, ntok):
    mesh = plsc.VectorSubcoreMesh(core_axis_name="c", subcore_axis_name="s")
    kern = functools.partial(
        pl.kernel, mesh=mesh,
        out_type=jax.ShapeDtypeStruct((ntok, N_DIM), jnp.float32),
        scratch_types=[
            pltpu.VMEM((_NCH, _CHUNK), jnp.int32),
            pltpu.VMEM((_GRP * HEADS * KNN,), jnp.float32),
            pltpu.VMEM((2, _CHUNK, _CH), jnp.float32),
            pltpu.VMEM((2, _GRP, _CH), jnp.float32),
            pltpu.SemaphoreType.DMA((2,)),
            pltpu.SemaphoreType.DMA((2,)),
        ],
    )(_make_emb_bag_body(ntok))
    return kern(values2, idx2, w)


_NSEG = 4


def kernel(x, Wq, keys1, keys2, values):
    keysC = jnp.concatenate([keys1, keys2], axis=0).astype(jnp.bfloat16)
    wq16 = Wq.astype(jnp.bfloat16)
    selA = jnp.asarray(_SELA)
    selB = jnp.asarray(_SELB)
    values2 = values.reshape(2 * values.shape[0], _CH)
    nseg_tok = N_TOK // _NSEG
    outs = []
    for sgi in range(_NSEG):
        xs = lax.slice_in_dim(x, sgi * nseg_tok, (sgi + 1) * nseg_tok, axis=0)
        idxT, wT = pl.pallas_call(
            _select_body,
            grid=(nseg_tok // TB,),
            in_specs=[pl.BlockSpec((HEADS * N_DIM, N_DIM), lambda i: (0, 0)),
                      pl.BlockSpec((2 * HEADS, N_KEYS, HALF),
                                   lambda i: (0, 0, 0)),
                      pl.BlockSpec((TB, N_DIM), lambda i: (i, 0)),
                      pl.BlockSpec((NCAND, KNN), lambda i: (0, 0)),
                      pl.BlockSpec((NCAND, KNN), lambda i: (0, 0))],
            out_specs=[pl.BlockSpec((HEADS * KNN, TB), lambda i: (0, i)),
                       pl.BlockSpec((HEADS * KNN, TB), lambda i: (0, i))],
            out_shape=[jax.ShapeDtypeStruct((HEADS * KNN, nseg_tok),
                                            jnp.float32),
                       jax.ShapeDtypeStruct((HEADS * KNN, nseg_tok),
                                            jnp.float32)],
        )(wq16, keysC, xs, selA, selB)
        flat_idx = idxT.T.astype(jnp.int32).reshape(nseg_tok * _CPT, _CHUNK)
        flat_w = wT.T.reshape(nseg_tok * HEADS * KNN)
        outs.append(_emb_bag(values2, flat_idx, flat_w, nseg_tok))
    return jnp.concatenate(outs, axis=0)


# speedup vs baseline: 1.4434x; 1.0396x over previous
import functools

import jax
import jax.numpy as jnp
import numpy as np
from jax import lax
from jax.experimental import pallas as pl
from jax.experimental.pallas import tpu as pltpu
from jax.experimental.pallas import tpu_sc as plsc

N_DIM = 512
N_KEYS = 512
HEADS = 4
KNN = 32
HALF = N_DIM // 2
N_TOK = 8192
TB = 128
NCAND = 128

_PAIRS = [(a, b) for a in range(KNN) for b in range(KNN) if (a + 1) * (b + 1) <= KNN]
_PAIRS = _PAIRS + [(KNN - 1, KNN - 1)] * (NCAND - len(_PAIRS))
_SELA = np.zeros((NCAND, KNN), np.float32)
_SELB = np.zeros((NCAND, KNN), np.float32)
for _c, (_a, _b) in enumerate(_PAIRS):
    _SELA[_c, _a] = 1.0
    _SELB[_c, _b] = 1.0


def _topk_T(s, k):
    n = s.shape[0]
    iota = lax.broadcasted_iota(jnp.int32, s.shape, 0)
    vals, idxs = [], []
    for _ in range(k):
        m = jnp.max(s, axis=0)
        sel = s == m[None, :]
        idx = jnp.min(jnp.where(sel, iota, n), axis=0)
        s = jnp.where(sel, -jnp.inf, s)
        vals.append(m)
        idxs.append(idx)
    return jnp.stack(vals), jnp.stack(idxs)


def _select_body(wq_ref, keys_ref, x_ref, selA_ref, selB_ref, idx_ref, w_ref):
    selA = selA_ref[...]
    selB = selB_ref[...]
    xb = x_ref[...].astype(jnp.bfloat16)
    hi = lax.Precision.HIGHEST
    qT = lax.dot_general(wq_ref[...], xb, (((1,), (1,)), ((), ())),
                         preferred_element_type=jnp.float32)
    qT16 = qT.astype(jnp.bfloat16)
    for u in range(HEADS):
        q1 = qT16[u * N_DIM: u * N_DIM + HALF, :]
        q2 = qT16[u * N_DIM + HALF: (u + 1) * N_DIM, :]
        sT1 = lax.dot_general(keys_ref[u], q1, (((1,), (0,)), ((), ())),
                              preferred_element_type=jnp.float32)
        sT2 = lax.dot_general(keys_ref[HEADS + u], q2, (((1,), (0,)), ((), ())),
                              preferred_element_type=jnp.float32)
        sc1, i1 = _topk_T(sT1, KNN)
        sc2, i2 = _topk_T(sT2, KNN)
        cand = (jnp.dot(selA, sc1, precision=hi)
                + jnp.dot(selB, sc2, precision=hi))
        ci1 = jnp.dot(selA, i1.astype(jnp.float32), precision=hi)
        ci2 = jnp.dot(selB, i2.astype(jnp.float32), precision=hi)
        cidx = ci1 * float(2 * N_KEYS) + ci2 * 2.0
        for k in range(KNN):
            m = jnp.max(cand, axis=0)
            sel = cand == m[None, :]
            iv = jnp.max(jnp.where(sel, cidx, -1.0), axis=0)
            cand = jnp.where(sel, -jnp.inf, cand)
            idx_ref[u * KNN + k, :] = iv
            w_ref[u * KNN + k, :] = jnp.maximum(m, 0.0)


_NC = 2
_NS = 16
_NW = _NC * _NS
_NTW = _NW // 2
_TPW = N_TOK // _NTW
_GRP = 16
_NGRP = _TPW // _GRP
_CHUNK = 64
_CPT = (HEADS * KNN) // _CHUNK
_NCH = _GRP * _CPT
_CH = N_DIM // 2
_CSL = _CH // 16


def _full16(v):
    return jnp.full((16,), v, jnp.int32)


def _make_emb_bag_body(ntok):
  _TPW = ntok // _NTW
  _NGRP = _TPW // _GRP

  def _emb_bag_body(values2, idx2, w, out, idx_v, w_v, rows_v, out_v, gsem, osem):
    wid = lax.axis_index("s") * _NC + lax.axis_index("c")
    gw = wid // 2
    chalf = wid % 2
    tok0_w = gw * _TPW
    chunk_row0 = gw * (_TPW * _CPT)

    def start(j):
        pltpu.async_copy(values2.at[idx_v.at[j]], rows_v.at[j % 2],
                         gsem.at[j % 2])

    def wait_g(j):
        pltpu.make_async_copy(values2.at[idx_v.at[j]], rows_v.at[j % 2],
                              gsem.at[j % 2]).wait()

    def group_body(g, _):
        pltpu.sync_copy(
            idx2.at[pl.ds(chunk_row0 + g * _NCH, _NCH)], idx_v)

        def fix_idx(r, carry):
            for q4 in range(_CHUNK // 16):
                sl = pl.ds(q4 * 16, 16)
                idx_v[r, sl] = idx_v[r, sl] + chalf
            return carry

        lax.fori_loop(0, _NCH, fix_idx, 0)
        pltpu.sync_copy(
            w.at[pl.ds((tok0_w + g * _GRP) * (HEADS * KNN),
                       _GRP * HEADS * KNN)], w_v)
        obuf = g % 2

        @pl.when(g >= 2)
        def _():
            pltpu.make_async_copy(
                out_v.at[obuf],
                out.at[pl.ds(tok0_w + (g - 2) * _GRP, _GRP),
                       pl.ds(chalf * _CH, _CH)],
                osem.at[obuf]).wait()

        start(0)
        start(1)

        def tok_body(t, _):
            dn = lax.GatherDimensionNumbers(
                offset_dims=(), collapsed_slice_dims=(0,),
                start_index_map=(0,))
            zero = jnp.zeros((16,), jnp.float32)
            for c in range(_CSL):
                out_v[obuf, t, pl.ds(c * 16, 16)] = zero
            for half in range(_CPT):
                j = _CPT * t + half
                wait_g(j)
                b = j % 2

                def blk_body(k16, carry, half=half, b=b):
                    wv16 = w_v[pl.ds(
                        t * (HEADS * KNN) + half * _CHUNK + k16 * 16, 16)]
                    acc = [zero] * _CSL
                    for jj in range(16):
                        wk = lax.gather(
                            wv16, _full16(jj)[:, None], dn, (1,),
                            mode=lax.GatherScatterMode.PROMISE_IN_BOUNDS)
                        k = k16 * 16 + jj
                        for c in range(_CSL):
                            acc[c] = acc[c] + wk * rows_v[b, k,
                                                          pl.ds(c * 16, 16)]
                    for c in range(_CSL):
                        sl = pl.ds(c * 16, 16)
                        out_v[obuf, t, sl] = out_v[obuf, t, sl] + acc[c]
                    return carry

                lax.fori_loop(0, _CHUNK // 16, blk_body, 0)

                @pl.when(j + 2 < _NCH)
                def _():
                    start(j + 2)
            return 0

        lax.fori_loop(0, _GRP, tok_body, 0)
        pltpu.async_copy(
            out_v.at[obuf],
            out.at[pl.ds(tok0_w + g * _GRP, _GRP), pl.ds(chalf * _CH, _CH)],
            osem.at[obuf])
        return 0

    lax.fori_loop(0, _NGRP, group_body, 0)
    for g in (_NGRP - 2, _NGRP - 1):
        pltpu.make_async_copy(
            out_v.at[g % 2],
            out.at[pl.ds(tok0_w + g * _GRP, _GRP), pl.ds(chalf * _CH, _CH)],
            osem.at[g % 2]).wait()

  return _emb_bag_body


def _emb_bag(values2, idx2, w, ntok):
    mesh = plsc.VectorSubcoreMesh(core_axis_name="c", subcore_axis_name="s")
    kern = functools.partial(
        pl.kernel, mesh=mesh,
        out_type=jax.ShapeDtypeStruct((ntok, N_DIM), jnp.float32),
        scratch_types=[
            pltpu.VMEM((_NCH, _CHUNK), jnp.int32),
            pltpu.VMEM((_GRP * HEADS * KNN,), jnp.float32),
            pltpu.VMEM((2, _CHUNK, _CH), jnp.float32),
            pltpu.VMEM((2, _GRP, _CH), jnp.float32),
            pltpu.SemaphoreType.DMA((2,)),
            pltpu.SemaphoreType.DMA((2,)),
        ],
    )(_make_emb_bag_body(ntok))
    return kern(values2, idx2, w)


_NSEG = 8


def kernel(x, Wq, keys1, keys2, values):
    keysC = jnp.concatenate([keys1, keys2], axis=0).astype(jnp.bfloat16)
    wq16 = Wq.astype(jnp.bfloat16)
    selA = jnp.asarray(_SELA)
    selB = jnp.asarray(_SELB)
    values2 = values.reshape(2 * values.shape[0], _CH)
    nseg_tok = N_TOK // _NSEG
    outs = []
    for sgi in range(_NSEG):
        xs = lax.slice_in_dim(x, sgi * nseg_tok, (sgi + 1) * nseg_tok, axis=0)
        idxT, wT = pl.pallas_call(
            _select_body,
            grid=(nseg_tok // TB,),
            in_specs=[pl.BlockSpec((HEADS * N_DIM, N_DIM), lambda i: (0, 0)),
                      pl.BlockSpec((2 * HEADS, N_KEYS, HALF),
                                   lambda i: (0, 0, 0)),
                      pl.BlockSpec((TB, N_DIM), lambda i: (i, 0)),
                      pl.BlockSpec((NCAND, KNN), lambda i: (0, 0)),
                      pl.BlockSpec((NCAND, KNN), lambda i: (0, 0))],
            out_specs=[pl.BlockSpec((HEADS * KNN, TB), lambda i: (0, i)),
                       pl.BlockSpec((HEADS * KNN, TB), lambda i: (0, i))],
            out_shape=[jax.ShapeDtypeStruct((HEADS * KNN, nseg_tok),
                                            jnp.float32),
                       jax.ShapeDtypeStruct((HEADS * KNN, nseg_tok),
                                            jnp.float32)],
        )(wq16, keysC, xs, selA, selB)
        flat_idx = idxT.T.astype(jnp.int32).reshape(nseg_tok * _CPT, _CHUNK)
        flat_w = wT.T.reshape(nseg_tok * HEADS * KNN)
        outs.append(_emb_bag(values2, flat_idx, flat_w, nseg_tok))
    return jnp.concatenate(outs, axis=0)
